# baseline (speedup 1.0000x reference)
import functools

import jax
import jax.numpy as jnp
from jax import lax
from jax.experimental import pallas as pl
from jax.experimental.pallas import tpu as pltpu
from jax.experimental.pallas import tpu_sc as plsc

N = 50000
F = 50000
K = 16
D_IN = 128
D = 64
NE = 4

NW = 32
FP = 51200
RPAIR = FP // 16
RF = 2560
RS = RPAIR - RF
FAST_C = 1
CROWS = 32
CH = CROWS * K
CPF = RF // CROWS
CPS = RS // CROWS
IDXW = 128
DROW = 64
GRP = CH // IDXW
SB = 320
FBROWS = RF * K // IDXW

BLK = 2000
GRID = F // BLK


def _k1_body(node_ref, hop_ref,
             wnm_ref, bnm_ref, wfm_ref, bfm_ref, wff_ref, bff_ref, wcat_ref,
             t_ref, nff0_ref):
    nnode = jnp.maximum(
        jnp.dot(node_ref[...], wnm_ref[...],
                preferred_element_type=jnp.float32) + bnm_ref[...], 0.0)
    t_ref[...] = jnp.dot(nnode, wcat_ref[...],
                         preferred_element_type=jnp.float32
                         ).astype(jnp.bfloat16)
    nhop = jnp.maximum(
        jnp.dot(hop_ref[...], wfm_ref[...],
                preferred_element_type=jnp.float32) + bfm_ref[...], 0.0)
    nff0_ref[...] = jnp.dot(nhop, wff_ref[...],
                            preferred_element_type=jnp.float32) + bff_ref[...]


def _k1(node, hop, W_nm, b_nm, W_fm, b_fm, W_f2f, b_f2f, Wcat):
    full = lambda arr: pl.BlockSpec(arr.shape, lambda i: (0, 0))
    return pl.pallas_call(
        _k1_body,
        grid=(GRID,),
        in_specs=[
            pl.BlockSpec((BLK, D_IN), lambda i: (i, 0)),
            pl.BlockSpec((BLK, D_IN), lambda i: (i, 0)),
            full(W_nm), full(b_nm), full(W_fm), full(b_fm),
            full(W_f2f), full(b_f2f), full(Wcat),
        ],
        out_specs=[
            pl.BlockSpec((BLK, NE * D), lambda i: (i, 0)),
            pl.BlockSpec((BLK, D), lambda i: (i, 0)),
        ],
        out_shape=[
            jax.ShapeDtypeStruct((N, NE * D), jnp.bfloat16),
            jax.ShapeDtypeStruct((F, D), jnp.float32),
        ],
    )(node, hop, W_nm, b_nm, W_fm, b_fm, W_f2f, b_f2f, Wcat)


def _sc_gather_mean(table, nn_idx, etype):
    mesh = plsc.VectorSubcoreMesh(core_axis_name="c", subcore_axis_name="s")

    @functools.partial(
        pl.kernel, mesh=mesh,
        out_type=jax.ShapeDtypeStruct((FP, D), jnp.float32),
        compiler_params=pltpu.CompilerParams(use_tc_tiling_on_sc=False,
                                             needs_layout_passes=False),
        scratch_types=[
            pltpu.VMEM((SB, K), jnp.int32),
            pltpu.VMEM((SB, K), jnp.int32),
            pltpu.VMEM((FBROWS, IDXW), jnp.int32),
            pltpu.VMEM((2, CH, DROW), jnp.bfloat16),
            pltpu.VMEM((2, CROWS, D), jnp.float32),
            pltpu.SemaphoreType.DMA,
            pltpu.SemaphoreType.DMA,
            pltpu.SemaphoreType.DMA,
            pltpu.SemaphoreType.DMA,
        ],
    )
    def k(table_hbm, nn_idx_hbm, et_hbm, out_hbm, idx_st, et_st, fidx_v,
          rows_v, acc_v, sg0, sg1, sw0, sw1):
        cax = lax.axis_index("c")
        s = lax.axis_index("s")
        is_fast = cax == FAST_C
        row0 = s * RPAIR + jnp.where(is_fast, 0, RF)
        nch = jnp.where(is_fast, CPF, CPS)

        def build(nrows):
            for b in range(nrows // SB):
                base = b * SB
                cb = jnp.clip(row0 + base, 0, F - SB)
                shift = row0 + base - cb
                pltpu.sync_copy(nn_idx_hbm.at[pl.ds(cb, SB)], idx_st)
                pltpu.sync_copy(et_hbm.at[pl.ds(cb, SB)], et_st)

                def fuse(r, _):
                    ri = jnp.minimum(r + shift, SB - 1)
                    f = idx_st[ri] * NE + et_st[ri]
                    f = jnp.where(row0 + base + r < F, f, 0)
                    rr = base + r
                    fidx_v[rr // 8, pl.ds((rr % 8) * K, K)] = f
                    return 0

                lax.fori_loop(0, SB, fuse, 0, unroll=False)

        @pl.when(is_fast)
        def _():
            build(RF)

        @pl.when(jnp.logical_not(is_fast))
        def _():
            build(RS)

        sgs = (sg0, sg1)
        sws = (sw0, sw1)

        def fire(c, slot):
            for g in range(GRP):
                pltpu.async_copy(
                    table_hbm.at[fidx_v.at[c * GRP + g]],
                    rows_v.at[slot].at[pl.ds(g * IDXW, IDXW)], sgs[slot])

        def drain(slot):
            for g in range(GRP):
                pltpu.make_async_copy(
                    table_hbm.at[fidx_v.at[0]],
                    rows_v.at[slot].at[pl.ds(g * IDXW, IDXW)],
                    sgs[slot]).wait()

        def wait_wb(slot):
            pltpu.make_async_copy(
                acc_v.at[slot], out_hbm.at[pl.ds(row0, CROWS)],
                sws[slot]).wait()

        def reduce_fire(c, slot):
            buf = rows_v.at[slot]

            def row(r, _):
                for half in range(2):
                    v0 = buf[r * K, pl.ds(half * 32, 32)]
                    a, b = plsc.unpack(v0,
                                       format=plsc.PackFormat.INTERLEAVED)
                    for kk in range(1, K):
                        vk = buf[r * K + kk, pl.ds(half * 32, 32)]
                        ak, bk = plsc.unpack(vk,
                                             format=plsc.PackFormat.INTERLEAVED)
                        a = a + ak
                        b = b + bk
                    acc_v[slot, r, pl.ds(half * 32, 16)] = a * (1.0 / K)
                    acc_v[slot, r, pl.ds(half * 32 + 16, 16)] = b * (1.0 / K)
                return 0

            lax.fori_loop(0, CROWS, row, 0, unroll=False)
            pltpu.async_copy(acc_v.at[slot],
                             out_hbm.at[pl.ds(row0 + c * CROWS, CROWS)],
                             sws[slot])

        fire(0, 0)
        fire(1, 1)

        def body(c2, _):
            c = c2 * 2
            for slot in range(2):
                drain(slot)

                @pl.when(c2 > 0)
                def _():
                    wait_wb(slot)

                reduce_fire(c + slot, slot)

                @pl.when(c + slot + 2 < nch)
                def _():
                    fire(c + slot + 2, slot)
            return 0

        lax.fori_loop(0, nch // 2, body, 0, unroll=False)
        wait_wb(0)
        wait_wb(1)

    return k(table, nn_idx, etype)


def _k3_body(agg_ref, nff0_ref, w1_ref, b1_ref, w2_ref, b2_ref,
             nff_ref, s1_ref, s2_ref):
    agg = agg_ref[...]
    h = jnp.maximum(jnp.dot(agg, w1_ref[...],
                            preferred_element_type=jnp.float32) + b1_ref[...],
                    0.0)
    nf = agg + jnp.dot(h, w2_ref[...],
                       preferred_element_type=jnp.float32) + b2_ref[...]
    nff = nff0_ref[...] + nf
    nff_ref[...] = nff
    s1c = jnp.sum(nff, axis=0, keepdims=True)
    s2c = lax.dot_general(nff, nff, (((0,), (0,)), ((), ())),
                          preferred_element_type=jnp.float32)
    i = pl.program_id(0)

    @pl.when(i == 0)
    def _():
        s1_ref[...] = s1c
        s2_ref[...] = s2c

    @pl.when(i > 0)
    def _():
        s1_ref[...] += s1c
        s2_ref[...] += s2c


def _k3(agg_pad, nff0, W1, b1, W2, b2):
    full = lambda arr: pl.BlockSpec(arr.shape, lambda i: (0, 0))
    return pl.pallas_call(
        _k3_body,
        grid=(GRID,),
        in_specs=[
            pl.BlockSpec((BLK, D), lambda i: (i, 0)),
            pl.BlockSpec((BLK, D), lambda i: (i, 0)),
            full(W1), full(b1), full(W2), full(b2),
        ],
        out_specs=[
            pl.BlockSpec((BLK, D), lambda i: (i, 0)),
            pl.BlockSpec((1, D), lambda i: (0, 0)),
            pl.BlockSpec((D, D), lambda i: (0, 0)),
        ],
        out_shape=[
            jax.ShapeDtypeStruct((F, D), jnp.float32),
            jax.ShapeDtypeStruct((1, D), jnp.float32),
            jax.ShapeDtypeStruct((D, D), jnp.float32),
        ],
    )(agg_pad, nff0, W1, b1, W2, b2)


def _k4_body(nff_ref, s1_ref, s2_ref, wc1_ref, bc1_ref, wc2_ref, bc2_ref,
             out_ref):
    wc1 = wc1_ref[...]
    m = s1_ref[...] * (1.0 / F)
    q = jnp.dot(m, wc1, preferred_element_type=jnp.float32)
    mu = q + bc1_ref[...]
    m2w = jnp.dot(s2_ref[...] * (1.0 / F), wc1,
                  preferred_element_type=jnp.float32)
    ex2 = jnp.sum(wc1 * m2w, axis=0, keepdims=True)
    var = ex2 - q * q
    a = lax.rsqrt(var + 1e-5)
    h = jnp.dot(nff_ref[...], wc1,
                preferred_element_type=jnp.float32) + bc1_ref[...]
    hn = jnp.maximum((h - mu) * a, 0.0)
    out_ref[...] = (jnp.dot(hn, wc2_ref[...],
                            preferred_element_type=jnp.float32)
                    + bc2_ref[0, 0])


def _k4(nff, s1, s2, Wc1, bc1, Wc2p, bc2r):
    full = lambda arr: pl.BlockSpec(arr.shape, lambda i: (0, 0))
    return pl.pallas_call(
        _k4_body,
        grid=(GRID,),
        in_specs=[
            pl.BlockSpec((BLK, D), lambda i: (i, 0)),
            full(s1), full(s2), full(Wc1), full(bc1), full(Wc2p),
            pl.BlockSpec(memory_space=pltpu.SMEM),
        ],
        out_specs=pl.BlockSpec((BLK, 1), lambda i: (i, 0)),
        out_shape=jax.ShapeDtypeStruct((F, 1), jnp.float32),
    )(nff, s1, s2, Wc1, bc1, Wc2p, bc2r)


def kernel(node_feature, hop_features_0, nn_idx_f2v_0, nn_idx_v2f_0,
           etype_f2v_0, etype_v2f_0,
           W_nm, b_nm, W_fm, b_fm, W_v2v, b_v2v, W_f2f, b_f2f,
           We_f2v, W1_f2v, b1_f2v, W2_f2v, b2_f2v,
           We_v2f, W1_v2f, b1_v2f, W2_v2f, b2_v2f,
           Wc1, bc1, Wc2, bc2):
    p = [0] * D
    for h in range(2):
        for i in range(16):
            p[32 * h + 2 * i] = 32 * h + i
            p[32 * h + 2 * i + 1] = 32 * h + 16 + i
    perm = jnp.asarray([64 * e + p[j] for e in range(NE) for j in range(D)])
    Wcat = jnp.transpose(We_v2f, (1, 0, 2)).reshape(D, NE * D)[:, perm]

    t2, nff0 = _k1(
        node_feature, hop_features_0,
        W_nm, b_nm.reshape(1, D), W_fm, b_fm.reshape(1, D),
        W_f2f, b_f2f.reshape(1, D), Wcat)

    agg = _sc_gather_mean(t2.reshape(NE * N, D),
                          nn_idx_v2f_0.astype(jnp.int32),
                          etype_v2f_0.astype(jnp.int32))

    nff, s1, s2 = _k3(agg, nff0, W1_v2f, b1_v2f.reshape(1, D),
                      W2_v2f, b2_v2f.reshape(1, D))

    return _k4(nff, s1, s2, Wc1, bc1.reshape(1, 128),
               Wc2, bc2.reshape(1, 1))


# speedup vs baseline: 1.0383x; 1.0383x over previous
import functools

import jax
import jax.numpy as jnp
from jax import lax
from jax.experimental import pallas as pl
from jax.experimental.pallas import tpu as pltpu
from jax.experimental.pallas import tpu_sc as plsc

N = 50000
F = 50000
K = 16
D_IN = 128
D = 64
NE = 4

NW = 32
FP = 51200
RPAIR = FP // 16
RF = 2240
RS = RPAIR - RF
FAST_C = 1
CROWS = 32
CH = CROWS * K
CPF = RF // CROWS
CPS = RS // CROWS
IDXW = 128
DROW = 64
GRP = CH // IDXW
SB = 320
FBROWS = RF * K // IDXW

BLK = 2000
GRID = F // BLK


def _k1_body(node_ref, hop_ref,
             wnm_ref, bnm_ref, wfm_ref, bfm_ref, wff_ref, bff_ref, wcat_ref,
             t_ref, nff0_ref):
    nnode = jnp.maximum(
        jnp.dot(node_ref[...], wnm_ref[...],
                preferred_element_type=jnp.float32) + bnm_ref[...], 0.0)
    t_ref[...] = jnp.dot(nnode, wcat_ref[...],
                         preferred_element_type=jnp.float32
                         ).astype(jnp.bfloat16)
    nhop = jnp.maximum(
        jnp.dot(hop_ref[...], wfm_ref[...],
                preferred_element_type=jnp.float32) + bfm_ref[...], 0.0)
    nff0_ref[...] = jnp.dot(nhop, wff_ref[...],
                            preferred_element_type=jnp.float32) + bff_ref[...]


def _k1(node, hop, W_nm, b_nm, W_fm, b_fm, W_f2f, b_f2f, Wcat):
    full = lambda arr: pl.BlockSpec(arr.shape, lambda i: (0, 0))
    return pl.pallas_call(
        _k1_body,
        grid=(GRID,),
        in_specs=[
            pl.BlockSpec((BLK, D_IN), lambda i: (i, 0)),
            pl.BlockSpec((BLK, D_IN), lambda i: (i, 0)),
            full(W_nm), full(b_nm), full(W_fm), full(b_fm),
            full(W_f2f), full(b_f2f), full(Wcat),
        ],
        out_specs=[
            pl.BlockSpec((BLK, NE * D), lambda i: (i, 0)),
            pl.BlockSpec((BLK, D), lambda i: (i, 0)),
        ],
        out_shape=[
            jax.ShapeDtypeStruct((N, NE * D), jnp.bfloat16),
            jax.ShapeDtypeStruct((F, D), jnp.float32),
        ],
    )(node, hop, W_nm, b_nm, W_fm, b_fm, W_f2f, b_f2f, Wcat)


def _sc_gather_mean(table, nn_idx, etype):
    mesh = plsc.VectorSubcoreMesh(core_axis_name="c", subcore_axis_name="s")

    @functools.partial(
        pl.kernel, mesh=mesh,
        out_type=jax.ShapeDtypeStruct((FP, D), jnp.float32),
        compiler_params=pltpu.CompilerParams(use_tc_tiling_on_sc=False,
                                             needs_layout_passes=False),
        scratch_types=[
            pltpu.VMEM((SB, K), jnp.int32),
            pltpu.VMEM((SB, K), jnp.int32),
            pltpu.VMEM((FBROWS, IDXW), jnp.int32),
            pltpu.VMEM((2, CH, DROW), jnp.bfloat16),
            pltpu.VMEM((2, CROWS, D), jnp.float32),
            pltpu.SemaphoreType.DMA,
            pltpu.SemaphoreType.DMA,
            pltpu.SemaphoreType.DMA,
            pltpu.SemaphoreType.DMA,
        ],
    )
    def k(table_hbm, nn_idx_hbm, et_hbm, out_hbm, idx_st, et_st, fidx_v,
          rows_v, acc_v, sg0, sg1, sw0, sw1):
        cax = lax.axis_index("c")
        s = lax.axis_index("s")
        is_fast = cax == FAST_C
        row0 = s * RPAIR + jnp.where(is_fast, 0, RF)
        nch = jnp.where(is_fast, CPF, CPS)

        def build(nrows):
            for b in range(nrows // SB):
                base = b * SB
                cb = jnp.clip(row0 + base, 0, F - SB)
                shift = row0 + base - cb
                pltpu.sync_copy(nn_idx_hbm.at[pl.ds(cb, SB)], idx_st)
                pltpu.sync_copy(et_hbm.at[pl.ds(cb, SB)], et_st)

                def fuse(r, _):
                    ri = jnp.minimum(r + shift, SB - 1)
                    f = idx_st[ri] * NE + et_st[ri]
                    f = jnp.where(row0 + base + r < F, f, 0)
                    rr = base + r
                    fidx_v[rr // 8, pl.ds((rr % 8) * K, K)] = f
                    return 0

                lax.fori_loop(0, SB, fuse, 0, unroll=False)

        @pl.when(is_fast)
        def _():
            build(RF)

        @pl.when(jnp.logical_not(is_fast))
        def _():
            build(RS)

        sgs = (sg0, sg1)
        sws = (sw0, sw1)

        def fire(c, slot):
            for g in range(GRP):
                pltpu.async_copy(
                    table_hbm.at[fidx_v.at[c * GRP + g]],
                    rows_v.at[slot].at[pl.ds(g * IDXW, IDXW)], sgs[slot])

        def drain(slot):
            for g in range(GRP):
                pltpu.make_async_copy(
                    table_hbm.at[fidx_v.at[0]],
                    rows_v.at[slot].at[pl.ds(g * IDXW, IDXW)],
                    sgs[slot]).wait()

        def wait_wb(slot):
            pltpu.make_async_copy(
                acc_v.at[slot], out_hbm.at[pl.ds(row0, CROWS)],
                sws[slot]).wait()

        def reduce_fire(c, slot):
            buf = rows_v.at[slot]

            def row(r, _):
                for half in range(2):
                    v0 = buf[r * K, pl.ds(half * 32, 32)]
                    a, b = plsc.unpack(v0,
                                       format=plsc.PackFormat.INTERLEAVED)
                    for kk in range(1, K):
                        vk = buf[r * K + kk, pl.ds(half * 32, 32)]
                        ak, bk = plsc.unpack(vk,
                                             format=plsc.PackFormat.INTERLEAVED)
                        a = a + ak
                        b = b + bk
                    acc_v[slot, r, pl.ds(half * 32, 16)] = a * (1.0 / K)
                    acc_v[slot, r, pl.ds(half * 32 + 16, 16)] = b * (1.0 / K)
                return 0

            lax.fori_loop(0, CROWS, row, 0, unroll=False)
            pltpu.async_copy(acc_v.at[slot],
                             out_hbm.at[pl.ds(row0 + c * CROWS, CROWS)],
                             sws[slot])

        fire(0, 0)
        fire(1, 1)

        def body(c2, _):
            c = c2 * 2
            for slot in range(2):
                drain(slot)

                @pl.when(c2 > 0)
                def _():
                    wait_wb(slot)

                reduce_fire(c + slot, slot)

                @pl.when(c + slot + 2 < nch)
                def _():
                    fire(c + slot + 2, slot)
            return 0

        lax.fori_loop(0, nch // 2, body, 0, unroll=False)
        wait_wb(0)
        wait_wb(1)

    return k(table, nn_idx, etype)


def _k3_body(agg_ref, nff0_ref, w1_ref, b1_ref, w2_ref, b2_ref,
             nff_ref, s1_ref, s2_ref):
    agg = agg_ref[...]
    h = jnp.maximum(jnp.dot(agg, w1_ref[...],
                            preferred_element_type=jnp.float32) + b1_ref[...],
                    0.0)
    nf = agg + jnp.dot(h, w2_ref[...],
                       preferred_element_type=jnp.float32) + b2_ref[...]
    nff = nff0_ref[...] + nf
    nff_ref[...] = nff
    s1c = jnp.sum(nff, axis=0, keepdims=True)
    s2c = lax.dot_general(nff, nff, (((0,), (0,)), ((), ())),
                          preferred_element_type=jnp.float32)
    i = pl.program_id(0)

    @pl.when(i == 0)
    def _():
        s1_ref[...] = s1c
        s2_ref[...] = s2c

    @pl.when(i > 0)
    def _():
        s1_ref[...] += s1c
        s2_ref[...] += s2c


def _k3(agg_pad, nff0, W1, b1, W2, b2):
    full = lambda arr: pl.BlockSpec(arr.shape, lambda i: (0, 0))
    return pl.pallas_call(
        _k3_body,
        grid=(GRID,),
        in_specs=[
            pl.BlockSpec((BLK, D), lambda i: (i, 0)),
            pl.BlockSpec((BLK, D), lambda i: (i, 0)),
            full(W1), full(b1), full(W2), full(b2),
        ],
        out_specs=[
            pl.BlockSpec((BLK, D), lambda i: (i, 0)),
            pl.BlockSpec((1, D), lambda i: (0, 0)),
            pl.BlockSpec((D, D), lambda i: (0, 0)),
        ],
        out_shape=[
            jax.ShapeDtypeStruct((F, D), jnp.float32),
            jax.ShapeDtypeStruct((1, D), jnp.float32),
            jax.ShapeDtypeStruct((D, D), jnp.float32),
        ],
    )(agg_pad, nff0, W1, b1, W2, b2)


def _k4_body(nff_ref, s1_ref, s2_ref, wc1_ref, bc1_ref, wc2_ref, bc2_ref,
             out_ref):
    wc1 = wc1_ref[...]
    m = s1_ref[...] * (1.0 / F)
    q = jnp.dot(m, wc1, preferred_element_type=jnp.float32)
    mu = q + bc1_ref[...]
    m2w = jnp.dot(s2_ref[...] * (1.0 / F), wc1,
                  preferred_element_type=jnp.float32)
    ex2 = jnp.sum(wc1 * m2w, axis=0, keepdims=True)
    var = ex2 - q * q
    a = lax.rsqrt(var + 1e-5)
    h = jnp.dot(nff_ref[...], wc1,
                preferred_element_type=jnp.float32) + bc1_ref[...]
    hn = jnp.maximum((h - mu) * a, 0.0)
    out_ref[...] = (jnp.dot(hn, wc2_ref[...],
                            preferred_element_type=jnp.float32)
                    + bc2_ref[0, 0])


def _k4(nff, s1, s2, Wc1, bc1, Wc2p, bc2r):
    full = lambda arr: pl.BlockSpec(arr.shape, lambda i: (0, 0))
    return pl.pallas_call(
        _k4_body,
        grid=(GRID,),
        in_specs=[
            pl.BlockSpec((BLK, D), lambda i: (i, 0)),
            full(s1), full(s2), full(Wc1), full(bc1), full(Wc2p),
            pl.BlockSpec(memory_space=pltpu.SMEM),
        ],
        out_specs=pl.BlockSpec((BLK, 1), lambda i: (i, 0)),
        out_shape=jax.ShapeDtypeStruct((F, 1), jnp.float32),
    )(nff, s1, s2, Wc1, bc1, Wc2p, bc2r)


def kernel(node_feature, hop_features_0, nn_idx_f2v_0, nn_idx_v2f_0,
           etype_f2v_0, etype_v2f_0,
           W_nm, b_nm, W_fm, b_fm, W_v2v, b_v2v, W_f2f, b_f2f,
           We_f2v, W1_f2v, b1_f2v, W2_f2v, b2_f2v,
           We_v2f, W1_v2f, b1_v2f, W2_v2f, b2_v2f,
           Wc1, bc1, Wc2, bc2):
    p = [0] * D
    for h in range(2):
        for i in range(16):
            p[32 * h + 2 * i] = 32 * h + i
            p[32 * h + 2 * i + 1] = 32 * h + 16 + i
    perm = jnp.asarray([64 * e + p[j] for e in range(NE) for j in range(D)])
    Wcat = jnp.transpose(We_v2f, (1, 0, 2)).reshape(D, NE * D)[:, perm]

    t2, nff0 = _k1(
        node_feature, hop_features_0,
        W_nm, b_nm.reshape(1, D), W_fm, b_fm.reshape(1, D),
        W_f2f, b_f2f.reshape(1, D), Wcat)

    agg = _sc_gather_mean(t2.reshape(NE * N, D),
                          nn_idx_v2f_0.astype(jnp.int32),
                          etype_v2f_0.astype(jnp.int32))

    nff, s1, s2 = _k3(agg, nff0, W1_v2f, b1_v2f.reshape(1, D),
                      W2_v2f, b2_v2f.reshape(1, D))

    return _k4(nff, s1, s2, Wc1, bc1.reshape(1, 128),
               Wc2, bc2.reshape(1, 1))
